# Initial kernel scaffold; baseline (speedup 1.0000x reference)
import jax
import jax.numpy as jnp
from jax.experimental import pallas as pl


def kernel(inputs, emb_table, W1, b1, W2, b2):
    raise NotImplementedError("write your pallas kernel here")


# speedup vs baseline: 10.7338x; 10.7338x over previous
import functools

import jax
import jax.numpy as jnp
from jax import lax
from jax.experimental import pallas as pl
from jax.experimental.pallas import tpu as pltpu
from jax.experimental.pallas import tpu_sc as plsc

BATCH = 16384
CTX = 6
EMBED_DIM = 128
OUTPUTS = 91

NC = 2
NS = 16
NW = NC * NS

N_IDX = BATCH * CTX
PER_W = N_IDX // NW
CHUNK = 128
ITERS = PER_W // CHUNK


def _sc_gather_body(idx_hbm, table_hbm, out_hbm, idx_v, buf, sem):
    c = lax.axis_index("c")
    s = lax.axis_index("s")
    wid = s * NC + c
    base = wid * PER_W
    pltpu.sync_copy(idx_hbm.at[wid], idx_v)

    def body(j, carry):
        pltpu.async_copy(table_hbm.at[idx_v.at[j]], buf, sem).wait()
        pltpu.sync_copy(buf, out_hbm.at[pl.ds(base + j * CHUNK, CHUNK)])
        return carry

    lax.fori_loop(0, ITERS, body, 0)


@jax.jit
def _sc_gather(idx3, table):
    mesh = plsc.VectorSubcoreMesh(core_axis_name="c", subcore_axis_name="s")
    return pl.kernel(
        _sc_gather_body,
        mesh=mesh,
        out_type=jax.ShapeDtypeStruct((N_IDX, EMBED_DIM), jnp.float32),
        scratch_types=[
            pltpu.VMEM((ITERS, CHUNK), jnp.int32),
            pltpu.VMEM((CHUNK, EMBED_DIM), jnp.float32),
            pltpu.SemaphoreType.DMA,
        ],
    )(idx3, table)


def _mlp_body(x_ref, w1_ref, b1_ref, w2_ref, b2_ref, o_ref):
    h = jnp.dot(x_ref[...], w1_ref[...], preferred_element_type=jnp.float32)
    h = jnp.maximum(h + b1_ref[...], 0.0)
    o_ref[...] = (
        jnp.dot(h, w2_ref[...], preferred_element_type=jnp.float32) + b2_ref[...]
    )


@functools.partial(jax.jit, static_argnames=("bm",))
def _mlp(x, w1, b1, w2, b2, bm=1024):
    nb = BATCH // bm
    return pl.pallas_call(
        _mlp_body,
        grid=(nb,),
        in_specs=[
            pl.BlockSpec((bm, CTX * EMBED_DIM), lambda i: (i, 0)),
            pl.BlockSpec((CTX * EMBED_DIM, EMBED_DIM), lambda i: (0, 0)),
            pl.BlockSpec((1, EMBED_DIM), lambda i: (0, 0)),
            pl.BlockSpec((EMBED_DIM, OUTPUTS), lambda i: (0, 0)),
            pl.BlockSpec((1, OUTPUTS), lambda i: (0, 0)),
        ],
        out_specs=pl.BlockSpec((bm, OUTPUTS), lambda i: (i, 0)),
        out_shape=jax.ShapeDtypeStruct((BATCH, OUTPUTS), jnp.float32),
    )(x, w1, b1, w2, b2)


def kernel(inputs, emb_table, W1, b1, W2, b2):
    idx3 = inputs.astype(jnp.int32).reshape(NW, ITERS, CHUNK)
    flat = _sc_gather(idx3, emb_table)
    x = flat.reshape(BATCH, CTX * EMBED_DIM)
    return _mlp(x, W1, b1.reshape(1, EMBED_DIM), W2, b2.reshape(1, OUTPUTS))


# speedup vs baseline: 11.7125x; 1.0912x over previous
import functools

import jax
import jax.numpy as jnp
from jax import lax
from jax.experimental import pallas as pl
from jax.experimental.pallas import tpu as pltpu
from jax.experimental.pallas import tpu_sc as plsc

BATCH = 16384
CTX = 6
EMBED_DIM = 128
OUTPUTS = 91

NC = 2
NS = 16
NW = NC * NS

N_IDX = BATCH * CTX
PER_W = N_IDX // NW
CHUNK = 128
ITERS = PER_W // CHUNK


NBUF = 4


def _sc_gather_body(idx_hbm, table_hbm, out_hbm, idx_v, bufs, gsem, ssem):
    c = lax.axis_index("c")
    s = lax.axis_index("s")
    wid = s * NC + c
    base = wid * PER_W
    pltpu.sync_copy(idx_hbm.at[wid], idx_v)

    def start_gather(b, j):
        pltpu.async_copy(table_hbm.at[idx_v.at[j]], bufs.at[b], gsem.at[b])

    def start_store(b, j):
        pltpu.async_copy(
            bufs.at[b], out_hbm.at[pl.ds(base + j * CHUNK, CHUNK)], ssem.at[b]
        )

    def wait_gather(b):
        pltpu.make_async_copy(
            table_hbm.at[idx_v.at[0]], bufs.at[b], gsem.at[b]
        ).wait()

    def wait_store(b):
        pltpu.make_async_copy(
            bufs.at[b], out_hbm.at[pl.ds(base, CHUNK)], ssem.at[b]
        ).wait()

    for b in range(NBUF):
        start_gather(b, b)

    def outer(t, carry):
        j0 = t * NBUF
        for b in range(NBUF):
            wait_gather(b)
            start_store(b, j0 + b)
        for b in range(NBUF):
            wait_store(b)
            start_gather(b, j0 + NBUF + b)
        return carry

    lax.fori_loop(0, ITERS // NBUF - 1, outer, 0)

    j0 = ITERS - NBUF
    for b in range(NBUF):
        wait_gather(b)
        start_store(b, j0 + b)
    for b in range(NBUF):
        wait_store(b)


@jax.jit
def _sc_gather(idx3, table):
    mesh = plsc.VectorSubcoreMesh(core_axis_name="c", subcore_axis_name="s")
    return pl.kernel(
        _sc_gather_body,
        mesh=mesh,
        out_type=jax.ShapeDtypeStruct((N_IDX, EMBED_DIM), jnp.float32),
        scratch_types=[
            pltpu.VMEM((ITERS, CHUNK), jnp.int32),
            pltpu.VMEM((NBUF, CHUNK, EMBED_DIM), jnp.float32),
            pltpu.SemaphoreType.DMA((NBUF,)),
            pltpu.SemaphoreType.DMA((NBUF,)),
        ],
    )(idx3, table)


def _mlp_body(x_ref, w1_ref, b1_ref, w2_ref, b2_ref, o_ref):
    h = jnp.dot(x_ref[...], w1_ref[...], preferred_element_type=jnp.float32)
    h = jnp.maximum(h + b1_ref[...], 0.0)
    o_ref[...] = (
        jnp.dot(h, w2_ref[...], preferred_element_type=jnp.float32) + b2_ref[...]
    )


@functools.partial(jax.jit, static_argnames=("bm",))
def _mlp(x, w1, b1, w2, b2, bm=1024):
    nb = BATCH // bm
    return pl.pallas_call(
        _mlp_body,
        grid=(nb,),
        in_specs=[
            pl.BlockSpec((bm, CTX * EMBED_DIM), lambda i: (i, 0)),
            pl.BlockSpec((CTX * EMBED_DIM, EMBED_DIM), lambda i: (0, 0)),
            pl.BlockSpec((1, EMBED_DIM), lambda i: (0, 0)),
            pl.BlockSpec((EMBED_DIM, OUTPUTS), lambda i: (0, 0)),
            pl.BlockSpec((1, OUTPUTS), lambda i: (0, 0)),
        ],
        out_specs=pl.BlockSpec((bm, OUTPUTS), lambda i: (i, 0)),
        out_shape=jax.ShapeDtypeStruct((BATCH, OUTPUTS), jnp.float32),
    )(x, w1, b1, w2, b2)


def kernel(inputs, emb_table, W1, b1, W2, b2):
    idx3 = inputs.astype(jnp.int32).reshape(NW, ITERS, CHUNK)
    flat = _sc_gather(idx3, emb_table)
    x = flat.reshape(BATCH, CTX * EMBED_DIM)
    return _mlp(x, W1, b1.reshape(1, EMBED_DIM), W2, b2.reshape(1, OUTPUTS))


# speedup vs baseline: 12.8637x; 1.0983x over previous
import functools

import jax
import jax.numpy as jnp
from jax import lax
from jax.experimental import pallas as pl
from jax.experimental.pallas import tpu as pltpu
from jax.experimental.pallas import tpu_sc as plsc

BATCH = 16384
CTX = 6
EMBED_DIM = 128
OUTPUTS = 91

NC = 2
NS = 16
NW = NC * NS

N_IDX = BATCH * CTX
PER_W = N_IDX // NW
CHUNK = 128
ITERS = PER_W // CHUNK
NBUF = 4


def _sc_gather_body(idx_hbm, table_hbm, out_hbm, idx_v, bufs, gsem, ssem):
    c = lax.axis_index("c")
    s = lax.axis_index("s")
    wid = s * NC + c
    base = wid * PER_W
    pltpu.sync_copy(idx_hbm.at[wid], idx_v)

    def start_gather(b, j):
        pltpu.async_copy(table_hbm.at[idx_v.at[j]], bufs.at[b], gsem.at[b])

    def start_store(b, j):
        pltpu.async_copy(
            bufs.at[b], out_hbm.at[pl.ds(base + j * CHUNK, CHUNK)], ssem.at[b]
        )

    def wait_gather(b):
        pltpu.make_async_copy(
            table_hbm.at[idx_v.at[0]], bufs.at[b], gsem.at[b]
        ).wait()

    def wait_store(b):
        pltpu.make_async_copy(
            bufs.at[b], out_hbm.at[pl.ds(base, CHUNK)], ssem.at[b]
        ).wait()

    for b in range(NBUF):
        start_gather(b, b)

    def outer(t, carry):
        j0 = t * NBUF
        for b in range(NBUF):
            wait_gather(b)
            start_store(b, j0 + b)
        for b in range(NBUF):
            wait_store(b)
            start_gather(b, j0 + NBUF + b)
        return carry

    lax.fori_loop(0, ITERS // NBUF - 1, outer, 0)

    j0 = ITERS - NBUF
    for b in range(NBUF):
        wait_gather(b)
        start_store(b, j0 + b)
    for b in range(NBUF):
        wait_store(b)


@jax.jit
def _sc_gather(idx3, table):
    mesh = plsc.VectorSubcoreMesh(core_axis_name="c", subcore_axis_name="s")
    return pl.kernel(
        _sc_gather_body,
        mesh=mesh,
        out_type=jax.ShapeDtypeStruct((N_IDX, EMBED_DIM), jnp.float32),
        scratch_types=[
            pltpu.VMEM((ITERS, CHUNK), jnp.int32),
            pltpu.VMEM((NBUF, CHUNK, EMBED_DIM), jnp.float32),
            pltpu.SemaphoreType.DMA((NBUF,)),
            pltpu.SemaphoreType.DMA((NBUF,)),
        ],
    )(idx3, table)


def _mlp_body(x_ref, w1_ref, b1_ref, w2_ref, b2_ref, o_ref, acc):
    k = pl.program_id(1)
    part = jnp.dot(x_ref[...], w1_ref[0], preferred_element_type=jnp.float32)

    @pl.when(k == 0)
    def _():
        acc[...] = part

    @pl.when(k > 0)
    def _():
        acc[...] += part

    @pl.when(k == CTX - 1)
    def _():
        h = jnp.maximum(acc[...] + b1_ref[...], 0.0)
        o_ref[...] = (
            jnp.dot(h, w2_ref[...], preferred_element_type=jnp.float32)
            + b2_ref[...]
        )


@functools.partial(jax.jit, static_argnames=("bm",))
def _mlp(xkm, w1k, b1, w2, b2, bm=1024):
    nb = BATCH // bm
    return pl.pallas_call(
        _mlp_body,
        grid=(nb, CTX),
        in_specs=[
            pl.BlockSpec((bm, EMBED_DIM), lambda i, k: (k * (BATCH // bm) + i, 0)),
            pl.BlockSpec((1, EMBED_DIM, EMBED_DIM), lambda i, k: (k, 0, 0)),
            pl.BlockSpec((1, EMBED_DIM), lambda i, k: (0, 0)),
            pl.BlockSpec((EMBED_DIM, OUTPUTS), lambda i, k: (0, 0)),
            pl.BlockSpec((1, OUTPUTS), lambda i, k: (0, 0)),
        ],
        out_specs=pl.BlockSpec((bm, OUTPUTS), lambda i, k: (i, 0)),
        out_shape=jax.ShapeDtypeStruct((BATCH, OUTPUTS), jnp.float32),
        scratch_shapes=[pltpu.VMEM((bm, EMBED_DIM), jnp.float32)],
    )(xkm, w1k, b1, w2, b2)


def kernel(inputs, emb_table, W1, b1, W2, b2):
    idx3 = inputs.astype(jnp.int32).T.reshape(NW, ITERS, CHUNK)
    xkm = _sc_gather(idx3, emb_table)
    w1k = W1.reshape(CTX, EMBED_DIM, EMBED_DIM)
    return _mlp(xkm, w1k, b1.reshape(1, EMBED_DIM), W2, b2.reshape(1, OUTPUTS))


# speedup vs baseline: 19.7143x; 1.5325x over previous
import functools

import jax
import jax.numpy as jnp
from jax import lax
from jax.experimental import pallas as pl
from jax.experimental.pallas import tpu as pltpu
from jax.experimental.pallas import tpu_sc as plsc

BATCH = 16384
CTX = 6
EMBED_DIM = 128
OUTPUTS = 91

NC = 2
NS = 16
NW = NC * NS

N_IDX = BATCH * CTX
PER_W = N_IDX // NW
CHUNK = 128
ITERS = PER_W // CHUNK
NBUF = 4


def _sc_gather_body(idx_hbm, table_hbm, out_hbm, idx_v, bufs, gsem, ssem):
    c = lax.axis_index("c")
    s = lax.axis_index("s")
    wid = s * NC + c
    base = wid * PER_W
    pltpu.sync_copy(idx_hbm.at[wid], idx_v)

    def start_gather(b, j):
        pltpu.async_copy(table_hbm.at[idx_v.at[j]], bufs.at[b], gsem.at[b])

    def start_store(b, j):
        pltpu.async_copy(
            bufs.at[b], out_hbm.at[pl.ds(base + j * CHUNK, CHUNK)], ssem.at[b]
        )

    def wait_gather(b):
        pltpu.make_async_copy(
            table_hbm.at[idx_v.at[0]], bufs.at[b], gsem.at[b]
        ).wait()

    def wait_store(b):
        pltpu.make_async_copy(
            bufs.at[b], out_hbm.at[pl.ds(base, CHUNK)], ssem.at[b]
        ).wait()

    for b in range(NBUF):
        start_gather(b, b)

    def outer(t, carry):
        j0 = t * NBUF
        for b in range(NBUF):
            wait_gather(b)
            start_store(b, j0 + b)
        for b in range(NBUF):
            wait_store(b)
            start_gather(b, j0 + NBUF + b)
        return carry

    lax.fori_loop(0, ITERS // NBUF - 1, outer, 0)

    j0 = ITERS - NBUF
    for b in range(NBUF):
        wait_gather(b)
        start_store(b, j0 + b)
    for b in range(NBUF):
        wait_store(b)


@jax.jit
def _sc_gather(idx3, table):
    mesh = plsc.VectorSubcoreMesh(core_axis_name="c", subcore_axis_name="s")
    return pl.kernel(
        _sc_gather_body,
        mesh=mesh,
        out_type=jax.ShapeDtypeStruct((N_IDX, EMBED_DIM), jnp.float32),
        scratch_types=[
            pltpu.VMEM((ITERS, CHUNK), jnp.int32),
            pltpu.VMEM((NBUF, CHUNK, EMBED_DIM), jnp.float32),
            pltpu.SemaphoreType.DMA((NBUF,)),
            pltpu.SemaphoreType.DMA((NBUF,)),
        ],
    )(idx3, table)


def _mlp_body(*refs):
    x_refs = refs[:CTX]
    w1_ref, b1_ref, w2_ref, b2_ref, o_ref = refs[CTX:]
    h = jnp.dot(x_refs[0][...], w1_ref[0], preferred_element_type=jnp.float32)
    for k in range(1, CTX):
        h += jnp.dot(x_refs[k][...], w1_ref[k], preferred_element_type=jnp.float32)
    h = jnp.maximum(h + b1_ref[...], 0.0)
    o_ref[...] = (
        jnp.dot(h, w2_ref[...], preferred_element_type=jnp.float32) + b2_ref[...]
    )


def _x_spec(k, bm):
    nb = BATCH // bm
    return pl.BlockSpec((bm, EMBED_DIM), lambda i, _k=k, _nb=nb: (_k * _nb + i, 0))


@functools.partial(jax.jit, static_argnames=("bm",))
def _mlp(xkm, w1k, b1, w2, b2, bm=1024):
    nb = BATCH // bm
    return pl.pallas_call(
        _mlp_body,
        grid=(nb,),
        in_specs=[_x_spec(k, bm) for k in range(CTX)]
        + [
            pl.BlockSpec((CTX, EMBED_DIM, EMBED_DIM), lambda i: (0, 0, 0)),
            pl.BlockSpec((1, EMBED_DIM), lambda i: (0, 0)),
            pl.BlockSpec((EMBED_DIM, OUTPUTS), lambda i: (0, 0)),
            pl.BlockSpec((1, OUTPUTS), lambda i: (0, 0)),
        ],
        out_specs=pl.BlockSpec((bm, OUTPUTS), lambda i: (i, 0)),
        out_shape=jax.ShapeDtypeStruct((BATCH, OUTPUTS), jnp.float32),
    )(*([xkm] * CTX), w1k, b1, w2, b2)


def kernel(inputs, emb_table, W1, b1, W2, b2):
    idx3 = inputs.astype(jnp.int32).T.reshape(NW, ITERS, CHUNK)
    xkm = _sc_gather(idx3, emb_table)
    w1k = W1.reshape(CTX, EMBED_DIM, EMBED_DIM)
    return _mlp(xkm, w1k, b1.reshape(1, EMBED_DIM), W2, b2.reshape(1, OUTPUTS))
